# Initial kernel scaffold; baseline (speedup 1.0000x reference)
import functools
import math

import jax
import jax.numpy as jnp
from jax import lax
from jax.experimental import pallas as pl
from jax.experimental.pallas import tpu as pltpu
from jax.experimental.pallas import tpu_sc as plsc

_NT = 32
_B = 128


@functools.cache
def _make_edge_kernel(N, NB):
    NR = N + 16
    RPT = NR // 16
    mesh = plsc.VectorSubcoreMesh(core_axis_name="c", subcore_axis_name="s")

    @functools.partial(
        pl.kernel,
        out_type=jax.ShapeDtypeStruct((2, NR, 32), jnp.float32),
        mesh=mesh,
        scratch_types=[
            pltpu.VMEM((_B,), jnp.int32),
            pltpu.VMEM((_B,), jnp.int32),
            pltpu.VMEM((_B, 16), jnp.float32),
            pltpu.VMEM((_B, 16), jnp.float32),
            pltpu.VMEM((_B, 32), jnp.float32),
            pltpu.VMEM((16,), jnp.float32),
            pltpu.VMEM((625, 32), jnp.float32),
            pltpu.VMEM_SHARED((10016, 32), jnp.float32),
            pltpu.SemaphoreType.DMA,
            pltpu.SemaphoreType.DMA,
        ],
    )
    def edge_kernel(xl_hbm, xr_hbm, src_hbm, dst_hbm, att_hbm, out_hbm,
                    src_v, dst_v, xlr, xrr, contrib, att_v, stage, acc,
                    sem1, sem2):
        cid = lax.axis_index("c")
        sid = lax.axis_index("s")
        wid = sid * 2 + cid
        RPT_ = 626

        z16 = jnp.zeros((16,), jnp.float32)

        def zrow(i, carry):
            stage[i, 0:16] = z16
            stage[i, 16:32] = z16
            return carry

        lax.fori_loop(0, RPT_, zrow, 0)
        pltpu.sync_copy(stage, acc.at[pl.ds(sid * RPT_, RPT_)])
        pltpu.sync_copy(att_hbm, att_v)
        plsc.subcore_barrier()

        attv = att_v[...]
        iota = lax.iota(jnp.int32, 16)

        def block(b, carry):
            pltpu.sync_copy(src_hbm.at[wid, b], src_v)
            pltpu.sync_copy(dst_hbm.at[wid, b], dst_v)
            cp1 = pltpu.async_copy(xl_hbm.at[src_v], xlr, sem1)
            cp2 = pltpu.async_copy(xr_hbm.at[dst_v], xrr, sem2)
            cp1.wait()
            cp2.wait()

            def edge(j, c):
                rl = xlr[j]
                a = rl + xrr[j]
                m = jnp.maximum(a, 0.2 * a)
                s = jnp.sum(m * attv)
                e = jnp.exp(lax.broadcast_in_dim(s, (16,), ()))
                contrib[j, 0:16] = e * rl
                contrib[j, 16:32] = jnp.where(iota == 0, e, 0.0)
                return c

            lax.fori_loop(0, _B, edge, 0)
            pltpu.sync_copy(contrib, acc.at[dst_v], add=True)
            return carry

        lax.fori_loop(0, NB, block, 0)
        plsc.subcore_barrier()

        pltpu.sync_copy(acc.at[pl.ds(sid * RPT_, RPT_)], stage)
        pltpu.sync_copy(stage, out_hbm.at[cid, pl.ds(sid * RPT_, RPT_)])

    return edge_kernel


def _lin_body(x_ref, wl_ref, wr_ref, xl_ref, xr_ref):
    xv = x_ref[...]
    xl_ref[...] = jnp.dot(xv, wl_ref[...], preferred_element_type=jnp.float32)
    xr_ref[...] = jnp.dot(xv, wr_ref[...], preferred_element_type=jnp.float32)


def _node_update(acc_ref, xl_ref, xr_ref, att_ref, b_ref, n):
    xl = xl_ref[...]
    xr = xr_ref[...]
    acc = acc_ref[...]
    num = acc[0, :n, 0:16] + acc[1, :n, 0:16]
    den = acc[0, :n, 16] + acc[1, :n, 16]
    att = att_ref[0, :]
    a = xl + xr
    m = jnp.maximum(a, 0.2 * a)
    e = jnp.exp(jnp.sum(m * att[None, :], axis=1))
    num = num + e[:, None] * xl
    den = den + e
    return num / (den[:, None] + 1e-16) + b_ref[0, :][None, :]


def _make_mid_body(n):
    def _mid_body(acc_ref, xl_ref, xr_ref, att_ref, b_ref, wl2_ref, wr2_ref,
                  xl2_ref, xr2_ref):
        h = _node_update(acc_ref, xl_ref, xr_ref, att_ref, b_ref, n)
        h = jnp.maximum(h, 0.0)
        xl2_ref[...] = jnp.dot(h, wl2_ref[...], preferred_element_type=jnp.float32)
        xr2_ref[...] = jnp.dot(h, wr2_ref[...], preferred_element_type=jnp.float32)
    return _mid_body


def _make_final_body(n):
    def _final_body(acc_ref, xl_ref, xr_ref, att_ref, b_ref, out_ref):
        h = _node_update(acc_ref, xl_ref, xr_ref, att_ref, b_ref, n)
        mx = jnp.max(h, axis=1, keepdims=True)
        t = h - mx
        out_ref[...] = t - jnp.log(jnp.sum(jnp.exp(t), axis=1, keepdims=True))
    return _final_body


def kernel(x, edge_index, Wl1, Wr1, att1, b1, Wl2, Wr2, att2, b2):
    N, _ = x.shape
    E = edge_index.shape[1]
    NB = math.ceil(E / (_NT * _B))
    E_pad = _NT * _B * NB

    src = jnp.concatenate(
        [edge_index[0], jnp.zeros((E_pad - E,), jnp.int32)]).reshape(_NT, NB, _B)
    dst = jnp.concatenate(
        [edge_index[1], jnp.full((E_pad - E,), N, jnp.int32)]).reshape(_NT, NB, _B)

    edge_kernel = _make_edge_kernel(N, NB)

    f32 = jnp.float32
    xl1, xr1 = pl.pallas_call(
        _lin_body,
        out_shape=[jax.ShapeDtypeStruct((N, 16), f32)] * 2,
    )(x, Wl1, Wr1)

    acc1 = edge_kernel(xl1, xr1, src, dst, att1)

    xl2, xr2 = pl.pallas_call(
        _make_mid_body(N),
        out_shape=[jax.ShapeDtypeStruct((N, 16), f32)] * 2,
    )(acc1, xl1, xr1, att1.reshape(1, 16), b1.reshape(1, 16), Wl2, Wr2)

    acc2 = edge_kernel(xl2, xr2, src, dst, att2)

    out = pl.pallas_call(
        _make_final_body(N),
        out_shape=jax.ShapeDtypeStruct((N, 16), f32),
    )(acc2, xl2, xr2, att2.reshape(1, 16), b2.reshape(1, 16))

    return out


# speedup vs baseline: 15.4840x; 15.4840x over previous
import functools
import math

import jax
import jax.numpy as jnp
from jax import lax
from jax.experimental import pallas as pl
from jax.experimental.pallas import tpu as pltpu
from jax.experimental.pallas import tpu_sc as plsc

_NT = 32
_B = 128


def _lane_shuffle(v, idx):
    dn = lax.GatherDimensionNumbers(
        offset_dims=(), collapsed_slice_dims=(0,), start_index_map=(0,))
    return lax.gather(v, idx[:, None], dn, (1,),
                      mode=lax.GatherScatterMode.PROMISE_IN_BOUNDS)


@functools.cache
def _make_edge_kernel(N, NB):
    RPT = math.ceil((N + 1) / 128) * 8
    NR = 16 * RPT
    mesh = plsc.VectorSubcoreMesh(core_axis_name="c", subcore_axis_name="s")

    @functools.partial(
        pl.kernel,
        out_type=jax.ShapeDtypeStruct((2, NR, 32), jnp.float32),
        mesh=mesh,
        compiler_params=pltpu.CompilerParams(use_tc_tiling_on_sc=False),
        scratch_types=[
            pltpu.VMEM((_B,), jnp.int32),
            pltpu.VMEM((_B,), jnp.int32),
            pltpu.VMEM((_B, 16), jnp.float32),
            pltpu.VMEM((_B, 16), jnp.float32),
            pltpu.VMEM((_B, 32), jnp.float32),
            pltpu.VMEM((16,), jnp.float32),
            pltpu.VMEM((RPT, 32), jnp.float32),
            pltpu.VMEM_SHARED((NR, 32), jnp.float32),
            pltpu.SemaphoreType.DMA,
            pltpu.SemaphoreType.DMA,
        ],
    )
    def edge_kernel(xl_hbm, xr_hbm, src_hbm, dst_hbm, att_hbm, out_hbm,
                    src_v, dst_v, xlr, xrr, contrib, att_v, stage, acc,
                    sem1, sem2):
        cid = lax.axis_index("c")
        sid = lax.axis_index("s")
        wid = sid * 2 + cid

        z16 = jnp.zeros((16,), jnp.float32)

        def zrow(i, carry):
            stage[i, 0:16] = z16
            stage[i, 16:32] = z16
            return carry

        lax.fori_loop(0, RPT, zrow, 0)
        pltpu.sync_copy(stage, acc.at[pl.ds(sid * RPT, RPT)])
        pltpu.sync_copy(att_hbm, att_v)
        plsc.subcore_barrier()

        attv = att_v[...]
        iota = lax.iota(jnp.int32, 16)
        perms = [iota ^ k for k in (8, 4, 2, 1)]

        def block(b, carry):
            pltpu.sync_copy(src_hbm.at[wid, b], src_v)
            pltpu.sync_copy(dst_hbm.at[wid, b], dst_v)
            cp1 = pltpu.async_copy(xl_hbm.at[src_v], xlr, sem1)
            cp2 = pltpu.async_copy(xr_hbm.at[dst_v], xrr, sem2)
            cp1.wait()
            cp2.wait()

            def edge(j, c):
                rl = xlr[j]
                a = rl + xrr[j]
                m = jnp.maximum(a, 0.2 * a)
                s = m * attv
                for p in perms:
                    s = s + _lane_shuffle(s, p)
                e = jnp.exp(s)
                contrib[j, 0:16] = e * rl
                contrib[j, 16:32] = jnp.where(iota == 0, e, 0.0)
                return c

            lax.fori_loop(0, _B, edge, 0)
            pltpu.sync_copy(contrib, acc.at[dst_v], add=True)
            return carry

        lax.fori_loop(0, NB, block, 0)
        plsc.subcore_barrier()

        pltpu.sync_copy(acc.at[pl.ds(sid * RPT, RPT)], stage)
        pltpu.sync_copy(stage, out_hbm.at[cid, pl.ds(sid * RPT, RPT)])

    return edge_kernel


def _lin_body(x_ref, wl_ref, wr_ref, xl_ref, xr_ref):
    xv = x_ref[...]
    xl_ref[...] = jnp.dot(xv, wl_ref[...], preferred_element_type=jnp.float32)
    xr_ref[...] = jnp.dot(xv, wr_ref[...], preferred_element_type=jnp.float32)


def _node_update(acc_ref, xl_ref, xr_ref, att_ref, b_ref, n):
    xl = xl_ref[...]
    xr = xr_ref[...]
    acc = acc_ref[...]
    num = acc[0, :n, 0:16] + acc[1, :n, 0:16]
    den = acc[0, :n, 16] + acc[1, :n, 16]
    att = att_ref[0, :]
    a = xl + xr
    m = jnp.maximum(a, 0.2 * a)
    e = jnp.exp(jnp.sum(m * att[None, :], axis=1))
    num = num + e[:, None] * xl
    den = den + e
    return num / (den[:, None] + 1e-16) + b_ref[0, :][None, :]


def _make_mid_body(n):
    def _mid_body(acc_ref, xl_ref, xr_ref, att_ref, b_ref, wl2_ref, wr2_ref,
                  xl2_ref, xr2_ref):
        h = _node_update(acc_ref, xl_ref, xr_ref, att_ref, b_ref, n)
        h = jnp.maximum(h, 0.0)
        xl2_ref[...] = jnp.dot(h, wl2_ref[...], preferred_element_type=jnp.float32)
        xr2_ref[...] = jnp.dot(h, wr2_ref[...], preferred_element_type=jnp.float32)
    return _mid_body


def _make_final_body(n):
    def _final_body(acc_ref, xl_ref, xr_ref, att_ref, b_ref, out_ref):
        h = _node_update(acc_ref, xl_ref, xr_ref, att_ref, b_ref, n)
        mx = jnp.max(h, axis=1, keepdims=True)
        t = h - mx
        out_ref[...] = t - jnp.log(jnp.sum(jnp.exp(t), axis=1, keepdims=True))
    return _final_body


def kernel(x, edge_index, Wl1, Wr1, att1, b1, Wl2, Wr2, att2, b2):
    N, _ = x.shape
    E = edge_index.shape[1]
    NB = math.ceil(E / (_NT * _B))
    E_pad = _NT * _B * NB

    src = jnp.concatenate(
        [edge_index[0], jnp.zeros((E_pad - E,), jnp.int32)]).reshape(_NT, NB, _B)
    dst = jnp.concatenate(
        [edge_index[1], jnp.full((E_pad - E,), N, jnp.int32)]).reshape(_NT, NB, _B)

    edge_kernel = _make_edge_kernel(N, NB)

    f32 = jnp.float32
    xl1, xr1 = pl.pallas_call(
        _lin_body,
        out_shape=[jax.ShapeDtypeStruct((N, 16), f32)] * 2,
    )(x, Wl1, Wr1)

    acc1 = edge_kernel(xl1, xr1, src, dst, att1)

    xl2, xr2 = pl.pallas_call(
        _make_mid_body(N),
        out_shape=[jax.ShapeDtypeStruct((N, 16), f32)] * 2,
    )(acc1, xl1, xr1, att1.reshape(1, 16), b1.reshape(1, 16), Wl2, Wr2)

    acc2 = edge_kernel(xl2, xr2, src, dst, att2)

    out = pl.pallas_call(
        _make_final_body(N),
        out_shape=jax.ShapeDtypeStruct((N, 16), f32),
    )(acc2, xl2, xr2, att2.reshape(1, 16), b2.reshape(1, 16))

    return out


# speedup vs baseline: 28.8821x; 1.8653x over previous
import functools
import math

import jax
import jax.numpy as jnp
from jax import lax
from jax.experimental import pallas as pl
from jax.experimental.pallas import tpu as pltpu
from jax.experimental.pallas import tpu_sc as plsc

_NT = 32
_B = 128


def _lane_shuffle(v, idx):
    dn = lax.GatherDimensionNumbers(
        offset_dims=(), collapsed_slice_dims=(0,), start_index_map=(0,))
    return lax.gather(v, idx[:, None], dn, (1,),
                      mode=lax.GatherScatterMode.PROMISE_IN_BOUNDS)


@functools.cache
def _make_edge_kernel(N, NB):
    RPT = math.ceil((N + 1) / 128) * 8
    NR = 16 * RPT
    mesh = plsc.VectorSubcoreMesh(core_axis_name="c", subcore_axis_name="s")

    @functools.partial(
        pl.kernel,
        out_type=jax.ShapeDtypeStruct((2, NR, 32), jnp.float32),
        mesh=mesh,
        compiler_params=pltpu.CompilerParams(use_tc_tiling_on_sc=False),
        scratch_types=[
            pltpu.VMEM((_B,), jnp.int32),
            pltpu.VMEM((_B,), jnp.int32),
            pltpu.VMEM((_B, 16), jnp.float32),
            pltpu.VMEM((_B, 16), jnp.float32),
            pltpu.VMEM((_B, 32), jnp.float32),
            pltpu.VMEM((16,), jnp.float32),
            pltpu.VMEM((RPT, 32), jnp.float32),
            pltpu.VMEM_SHARED((NR, 32), jnp.float32),
            pltpu.SemaphoreType.DMA,
            pltpu.SemaphoreType.DMA,
        ],
    )
    def edge_kernel(xl_hbm, xr_hbm, src_hbm, dst_hbm, att_hbm, out_hbm,
                    src_v, dst_v, xlr, xrr, contrib, att_v, stage, acc,
                    sem1, sem2):
        cid = lax.axis_index("c")
        sid = lax.axis_index("s")
        wid = sid * 2 + cid

        z16 = jnp.zeros((16,), jnp.float32)

        def zrow(i, carry):
            stage[i, 0:16] = z16
            stage[i, 16:32] = z16
            return carry

        lax.fori_loop(0, RPT, zrow, 0)
        pltpu.sync_copy(stage, acc.at[pl.ds(sid * RPT, RPT)])
        pltpu.sync_copy(att_hbm, att_v)
        plsc.subcore_barrier()

        attv = att_v[...]
        iota = lax.iota(jnp.int32, 16)
        perms = [iota ^ k for k in (8, 4, 2, 1)]

        def block(b, carry):
            pltpu.sync_copy(src_hbm.at[wid, b], src_v)
            pltpu.sync_copy(dst_hbm.at[wid, b], dst_v)
            cp1 = pltpu.async_copy(xl_hbm.at[src_v], xlr, sem1)
            cp2 = pltpu.async_copy(xr_hbm.at[dst_v], xrr, sem2)
            cp1.wait()
            cp2.wait()

            @plsc.parallel_loop(0, _B, 1, unroll=8)
            def edge(j):
                rl = xlr[j]
                a = rl + xrr[j]
                m = jnp.maximum(a, 0.2 * a)
                s = m * attv
                for p in perms:
                    s = s + _lane_shuffle(s, p)
                e = jnp.exp(s)
                contrib[j, 0:16] = e * rl
                contrib[j, 16:32] = jnp.where(iota == 0, e, 0.0)
            pltpu.sync_copy(contrib, acc.at[dst_v], add=True)
            return carry

        lax.fori_loop(0, NB, block, 0)
        plsc.subcore_barrier()

        pltpu.sync_copy(acc.at[pl.ds(sid * RPT, RPT)], stage)
        pltpu.sync_copy(stage, out_hbm.at[cid, pl.ds(sid * RPT, RPT)])

    return edge_kernel


def _lin_body(x_ref, wl_ref, wr_ref, xl_ref, xr_ref):
    xv = x_ref[...]
    xl_ref[...] = jnp.dot(xv, wl_ref[...], preferred_element_type=jnp.float32)
    xr_ref[...] = jnp.dot(xv, wr_ref[...], preferred_element_type=jnp.float32)


def _node_update(acc_ref, xl_ref, xr_ref, att_ref, b_ref, n):
    xl = xl_ref[...]
    xr = xr_ref[...]
    acc = acc_ref[...]
    num = acc[0, :n, 0:16] + acc[1, :n, 0:16]
    den = acc[0, :n, 16] + acc[1, :n, 16]
    att = att_ref[0, :]
    a = xl + xr
    m = jnp.maximum(a, 0.2 * a)
    e = jnp.exp(jnp.sum(m * att[None, :], axis=1))
    num = num + e[:, None] * xl
    den = den + e
    return num / (den[:, None] + 1e-16) + b_ref[0, :][None, :]


def _make_mid_body(n):
    def _mid_body(acc_ref, xl_ref, xr_ref, att_ref, b_ref, wl2_ref, wr2_ref,
                  xl2_ref, xr2_ref):
        h = _node_update(acc_ref, xl_ref, xr_ref, att_ref, b_ref, n)
        h = jnp.maximum(h, 0.0)
        xl2_ref[...] = jnp.dot(h, wl2_ref[...], preferred_element_type=jnp.float32)
        xr2_ref[...] = jnp.dot(h, wr2_ref[...], preferred_element_type=jnp.float32)
    return _mid_body


def _make_final_body(n):
    def _final_body(acc_ref, xl_ref, xr_ref, att_ref, b_ref, out_ref):
        h = _node_update(acc_ref, xl_ref, xr_ref, att_ref, b_ref, n)
        mx = jnp.max(h, axis=1, keepdims=True)
        t = h - mx
        out_ref[...] = t - jnp.log(jnp.sum(jnp.exp(t), axis=1, keepdims=True))
    return _final_body


def kernel(x, edge_index, Wl1, Wr1, att1, b1, Wl2, Wr2, att2, b2):
    N, _ = x.shape
    E = edge_index.shape[1]
    NB = math.ceil(E / (_NT * _B))
    E_pad = _NT * _B * NB

    src = jnp.concatenate(
        [edge_index[0], jnp.zeros((E_pad - E,), jnp.int32)]).reshape(_NT, NB, _B)
    dst = jnp.concatenate(
        [edge_index[1], jnp.full((E_pad - E,), N, jnp.int32)]).reshape(_NT, NB, _B)

    edge_kernel = _make_edge_kernel(N, NB)

    f32 = jnp.float32
    xl1, xr1 = pl.pallas_call(
        _lin_body,
        out_shape=[jax.ShapeDtypeStruct((N, 16), f32)] * 2,
    )(x, Wl1, Wr1)

    acc1 = edge_kernel(xl1, xr1, src, dst, att1)

    xl2, xr2 = pl.pallas_call(
        _make_mid_body(N),
        out_shape=[jax.ShapeDtypeStruct((N, 16), f32)] * 2,
    )(acc1, xl1, xr1, att1.reshape(1, 16), b1.reshape(1, 16), Wl2, Wr2)

    acc2 = edge_kernel(xl2, xr2, src, dst, att2)

    out = pl.pallas_call(
        _make_final_body(N),
        out_shape=jax.ShapeDtypeStruct((N, 16), f32),
    )(acc2, xl2, xr2, att2.reshape(1, 16), b2.reshape(1, 16))

    return out


# speedup vs baseline: 34.7611x; 1.2036x over previous
import functools
import math

import jax
import jax.numpy as jnp
from jax import lax
from jax.experimental import pallas as pl
from jax.experimental.pallas import tpu as pltpu
from jax.experimental.pallas import tpu_sc as plsc

_NT = 32
_B = 128


def _lane_shuffle(v, idx):
    dn = lax.GatherDimensionNumbers(
        offset_dims=(), collapsed_slice_dims=(0,), start_index_map=(0,))
    return lax.gather(v, idx[:, None], dn, (1,),
                      mode=lax.GatherScatterMode.PROMISE_IN_BOUNDS)


@functools.cache
def _make_edge_kernel(N, NB):
    RPT = math.ceil((N + 1) / 128) * 8
    NR = 16 * RPT
    mesh = plsc.VectorSubcoreMesh(core_axis_name="c", subcore_axis_name="s")

    @functools.partial(
        pl.kernel,
        out_type=jax.ShapeDtypeStruct((2, NR, 32), jnp.float32),
        mesh=mesh,
        compiler_params=pltpu.CompilerParams(use_tc_tiling_on_sc=False),
        scratch_types=[
            pltpu.VMEM((4, _B), jnp.int32),
            pltpu.VMEM((_B, 16), jnp.float32),
            pltpu.VMEM((_B, 16), jnp.float32),
            pltpu.VMEM((_B, 16), jnp.float32),
            pltpu.VMEM((_B, 16), jnp.float32),
            pltpu.VMEM((_B, 32), jnp.float32),
            pltpu.VMEM((_B, 32), jnp.float32),
            pltpu.VMEM((16,), jnp.float32),
            pltpu.VMEM((RPT, 32), jnp.float32),
            pltpu.VMEM_SHARED((NR, 32), jnp.float32),
            pltpu.SemaphoreType.DMA,
            pltpu.SemaphoreType.DMA,
            pltpu.SemaphoreType.DMA,
            pltpu.SemaphoreType.DMA,
        ],
    )
    def edge_kernel(xl_hbm, xr_hbm, idx_hbm, att_hbm, out_hbm,
                    idx_v, xlr0, xrr0, xlr1, xrr1,
                    contrib0, contrib1, att_v, stage, acc,
                    sxl0, sxr0, sxl1, sxr1):
        cid = lax.axis_index("c")
        sid = lax.axis_index("s")
        wid = sid * 2 + cid

        z16 = jnp.zeros((16,), jnp.float32)

        def zrow(i, carry):
            stage[i, 0:16] = z16
            stage[i, 16:32] = z16
            return carry

        lax.fori_loop(0, RPT, zrow, 0)
        pltpu.sync_copy(stage, acc.at[pl.ds(sid * RPT, RPT)])
        pltpu.sync_copy(att_hbm, att_v)
        plsc.subcore_barrier()

        attv = att_v[...]
        iota = lax.iota(jnp.int32, 16)
        perms = [iota ^ k for k in (8, 4, 2, 1)]

        def compute_block(xlr, xrr, contrib):
            @plsc.parallel_loop(0, _B, 1, unroll=8)
            def edge(j):
                rl = xlr[j]
                a = rl + xrr[j]
                m = jnp.maximum(a, 0.2 * a)
                s = m * attv
                for p in perms:
                    s = s + _lane_shuffle(s, p)
                e = jnp.exp(s)
                contrib[j, 0:16] = e * rl
                contrib[j, 16:32] = jnp.where(iota == 0, e, 0.0)

        NBS = NB // 2

        @pl.loop(0, NBS, step=1)
        def _blocks(sb):
            pltpu.sync_copy(idx_hbm.at[wid, sb], idx_v)
            g0l = pltpu.async_copy(xl_hbm.at[idx_v.at[0]], xlr0, sxl0)
            g0r = pltpu.async_copy(xr_hbm.at[idx_v.at[1]], xrr0, sxr0)
            g0l.wait()
            g0r.wait()
            g1l = pltpu.async_copy(xl_hbm.at[idx_v.at[2]], xlr1, sxl1)
            g1r = pltpu.async_copy(xr_hbm.at[idx_v.at[3]], xrr1, sxr1)
            compute_block(xlr0, xrr0, contrib0)
            g1l.wait()
            g1r.wait()
            compute_block(xlr1, xrr1, contrib1)
            pltpu.sync_copy(contrib0, acc.at[idx_v.at[1]], add=True)
            pltpu.sync_copy(contrib1, acc.at[idx_v.at[3]], add=True)

        plsc.subcore_barrier()

        pltpu.sync_copy(acc.at[pl.ds(sid * RPT, RPT)], stage)
        pltpu.sync_copy(stage, out_hbm.at[cid, pl.ds(sid * RPT, RPT)])

    return edge_kernel


def _lin_body(x_ref, wl_ref, wr_ref, xl_ref, xr_ref):
    xv = x_ref[...]
    xl_ref[...] = jnp.dot(xv, wl_ref[...], preferred_element_type=jnp.float32)
    xr_ref[...] = jnp.dot(xv, wr_ref[...], preferred_element_type=jnp.float32)


def _node_update(acc_ref, xl_ref, xr_ref, att_ref, b_ref, n):
    xl = xl_ref[...]
    xr = xr_ref[...]
    acc = acc_ref[...]
    num = acc[0, :n, 0:16] + acc[1, :n, 0:16]
    den = acc[0, :n, 16] + acc[1, :n, 16]
    att = att_ref[0, :]
    a = xl + xr
    m = jnp.maximum(a, 0.2 * a)
    e = jnp.exp(jnp.sum(m * att[None, :], axis=1))
    num = num + e[:, None] * xl
    den = den + e
    return num / (den[:, None] + 1e-16) + b_ref[0, :][None, :]


def _make_mid_body(n):
    def _mid_body(acc_ref, xl_ref, xr_ref, att_ref, b_ref, wl2_ref, wr2_ref,
                  xl2_ref, xr2_ref):
        h = _node_update(acc_ref, xl_ref, xr_ref, att_ref, b_ref, n)
        h = jnp.maximum(h, 0.0)
        xl2_ref[...] = jnp.dot(h, wl2_ref[...], preferred_element_type=jnp.float32)
        xr2_ref[...] = jnp.dot(h, wr2_ref[...], preferred_element_type=jnp.float32)
    return _mid_body


def _make_final_body(n):
    def _final_body(acc_ref, xl_ref, xr_ref, att_ref, b_ref, out_ref):
        h = _node_update(acc_ref, xl_ref, xr_ref, att_ref, b_ref, n)
        mx = jnp.max(h, axis=1, keepdims=True)
        t = h - mx
        out_ref[...] = t - jnp.log(jnp.sum(jnp.exp(t), axis=1, keepdims=True))
    return _final_body


def kernel(x, edge_index, Wl1, Wr1, att1, b1, Wl2, Wr2, att2, b2):
    N, _ = x.shape
    E = edge_index.shape[1]
    NB = 2 * math.ceil(E / (_NT * _B * 2))
    E_pad = _NT * _B * NB

    src = jnp.concatenate(
        [edge_index[0], jnp.zeros((E_pad - E,), jnp.int32)]).reshape(_NT, NB, _B)
    dst = jnp.concatenate(
        [edge_index[1], jnp.full((E_pad - E,), N, jnp.int32)]).reshape(_NT, NB, _B)
    s4 = src.reshape(_NT, NB // 2, 2, 1, _B)
    d4 = dst.reshape(_NT, NB // 2, 2, 1, _B)
    packed = jnp.concatenate([s4, d4], axis=3).reshape(_NT, NB // 2, 4, _B)

    edge_kernel = _make_edge_kernel(N, NB)

    f32 = jnp.float32
    xl1, xr1 = pl.pallas_call(
        _lin_body,
        out_shape=[jax.ShapeDtypeStruct((N, 16), f32)] * 2,
    )(x, Wl1, Wr1)

    acc1 = edge_kernel(xl1, xr1, packed, att1)

    xl2, xr2 = pl.pallas_call(
        _make_mid_body(N),
        out_shape=[jax.ShapeDtypeStruct((N, 16), f32)] * 2,
    )(acc1, xl1, xr1, att1.reshape(1, 16), b1.reshape(1, 16), Wl2, Wr2)

    acc2 = edge_kernel(xl2, xr2, packed, att2)

    out = pl.pallas_call(
        _make_final_body(N),
        out_shape=jax.ShapeDtypeStruct((N, 16), f32),
    )(acc2, xl2, xr2, att2.reshape(1, 16), b2.reshape(1, 16))

    return out


# speedup vs baseline: 37.3902x; 1.0756x over previous
import functools
import math

import jax
import jax.numpy as jnp
from jax import lax
from jax.experimental import pallas as pl
from jax.experimental.pallas import tpu as pltpu
from jax.experimental.pallas import tpu_sc as plsc

_NT = 32
_B = 128


def _lane_shuffle(v, idx):
    dn = lax.GatherDimensionNumbers(
        offset_dims=(), collapsed_slice_dims=(0,), start_index_map=(0,))
    return lax.gather(v, idx[:, None], dn, (1,),
                      mode=lax.GatherScatterMode.PROMISE_IN_BOUNDS)


@functools.cache
def _make_edge_kernel(N, NB):
    RPT = math.ceil((N + 1) / 128) * 8
    NR = 16 * RPT
    mesh = plsc.VectorSubcoreMesh(core_axis_name="c", subcore_axis_name="s")

    @functools.partial(
        pl.kernel,
        out_type=jax.ShapeDtypeStruct((2, NR, 32), jnp.float32),
        mesh=mesh,
        compiler_params=pltpu.CompilerParams(use_tc_tiling_on_sc=False),
        scratch_types=[
            pltpu.VMEM((8, _B), jnp.int32),
            pltpu.VMEM((_B, 16), jnp.float32),
            pltpu.VMEM((_B, 16), jnp.float32),
            pltpu.VMEM((_B, 16), jnp.float32),
            pltpu.VMEM((_B, 16), jnp.float32),
            pltpu.VMEM((_B, 32), jnp.float32),
            pltpu.VMEM((_B, 32), jnp.float32),
            pltpu.VMEM((_B, 32), jnp.float32),
            pltpu.VMEM((_B, 32), jnp.float32),
            pltpu.VMEM((16,), jnp.float32),
            pltpu.VMEM((RPT, 32), jnp.float32),
            pltpu.VMEM_SHARED((NR, 32), jnp.float32),
            pltpu.SemaphoreType.DMA,
            pltpu.SemaphoreType.DMA,
            pltpu.SemaphoreType.DMA,
            pltpu.SemaphoreType.DMA,
        ],
    )
    def edge_kernel(xl_hbm, xr_hbm, idx_hbm, att_hbm, out_hbm,
                    idx_v, xlr0, xrr0, xlr1, xrr1,
                    contrib0, contrib1, contrib2, contrib3, att_v, stage, acc,
                    sxl0, sxr0, sxl1, sxr1):
        cid = lax.axis_index("c")
        sid = lax.axis_index("s")
        wid = sid * 2 + cid

        z16 = jnp.zeros((16,), jnp.float32)

        def zrow(i, carry):
            stage[i, 0:16] = z16
            stage[i, 16:32] = z16
            return carry

        lax.fori_loop(0, RPT, zrow, 0)
        pltpu.sync_copy(stage, acc.at[pl.ds(sid * RPT, RPT)])
        pltpu.sync_copy(att_hbm, att_v)
        plsc.subcore_barrier()

        attv = att_v[...]
        iota = lax.iota(jnp.int32, 16)
        perms = [iota ^ k for k in (8, 4, 2, 1)]

        def compute_block(xlr, xrr, contrib):
            @plsc.parallel_loop(0, _B, 1, unroll=8)
            def edge(j):
                rl = xlr[j]
                a = rl + xrr[j]
                m = jnp.maximum(a, 0.2 * a)
                s = m * attv
                for p in perms:
                    s = s + _lane_shuffle(s, p)
                e = jnp.exp(s)
                contrib[j, 0:16] = e * rl
                contrib[j, 16:32] = jnp.where(iota == 0, e, 0.0)

        NBS = NB // 4
        xlrs = (xlr0, xlr1)
        xrrs = (xrr0, xrr1)
        contribs = (contrib0, contrib1, contrib2, contrib3)
        sxls = (sxl0, sxl1)
        sxrs = (sxr0, sxr1)

        @pl.loop(0, NBS, step=1)
        def _blocks(sb):
            pltpu.sync_copy(idx_hbm.at[wid, sb], idx_v)
            g = (pltpu.async_copy(xl_hbm.at[idx_v.at[0]], xlr0, sxl0),
                 pltpu.async_copy(xr_hbm.at[idx_v.at[1]], xrr0, sxr0))
            for blk in range(4):
                slot = blk % 2
                nslot = 1 - slot
                g[0].wait()
                g[1].wait()
                if blk < 3:
                    g = (pltpu.async_copy(
                            xl_hbm.at[idx_v.at[2 * blk + 2]],
                            xlrs[nslot], sxls[nslot]),
                         pltpu.async_copy(
                            xr_hbm.at[idx_v.at[2 * blk + 3]],
                            xrrs[nslot], sxrs[nslot]))
                compute_block(xlrs[slot], xrrs[slot], contribs[blk])
            for blk in range(4):
                pltpu.sync_copy(
                    contribs[blk], acc.at[idx_v.at[2 * blk + 1]], add=True)

        plsc.subcore_barrier()

        pltpu.sync_copy(acc.at[pl.ds(sid * RPT, RPT)], stage)
        pltpu.sync_copy(stage, out_hbm.at[cid, pl.ds(sid * RPT, RPT)])

    return edge_kernel


def _lin_body(x_ref, wl_ref, wr_ref, xl_ref, xr_ref):
    xv = x_ref[...]
    xl_ref[...] = jnp.dot(xv, wl_ref[...], preferred_element_type=jnp.float32)
    xr_ref[...] = jnp.dot(xv, wr_ref[...], preferred_element_type=jnp.float32)


def _node_update(acc_ref, xl_ref, xr_ref, att_ref, b_ref, n):
    xl = xl_ref[...]
    xr = xr_ref[...]
    acc = acc_ref[...]
    num = acc[0, :n, 0:16] + acc[1, :n, 0:16]
    den = acc[0, :n, 16] + acc[1, :n, 16]
    att = att_ref[0, :]
    a = xl + xr
    m = jnp.maximum(a, 0.2 * a)
    e = jnp.exp(jnp.sum(m * att[None, :], axis=1))
    num = num + e[:, None] * xl
    den = den + e
    return num / (den[:, None] + 1e-16) + b_ref[0, :][None, :]


def _make_mid_body(n):
    def _mid_body(acc_ref, xl_ref, xr_ref, att_ref, b_ref, wl2_ref, wr2_ref,
                  xl2_ref, xr2_ref):
        h = _node_update(acc_ref, xl_ref, xr_ref, att_ref, b_ref, n)
        h = jnp.maximum(h, 0.0)
        xl2_ref[...] = jnp.dot(h, wl2_ref[...], preferred_element_type=jnp.float32)
        xr2_ref[...] = jnp.dot(h, wr2_ref[...], preferred_element_type=jnp.float32)
    return _mid_body


def _make_final_body(n):
    def _final_body(acc_ref, xl_ref, xr_ref, att_ref, b_ref, out_ref):
        h = _node_update(acc_ref, xl_ref, xr_ref, att_ref, b_ref, n)
        mx = jnp.max(h, axis=1, keepdims=True)
        t = h - mx
        out_ref[...] = t - jnp.log(jnp.sum(jnp.exp(t), axis=1, keepdims=True))
    return _final_body


def kernel(x, edge_index, Wl1, Wr1, att1, b1, Wl2, Wr2, att2, b2):
    N, _ = x.shape
    E = edge_index.shape[1]
    NB = 4 * math.ceil(E / (_NT * _B * 4))
    E_pad = _NT * _B * NB

    src = jnp.concatenate(
        [edge_index[0], jnp.zeros((E_pad - E,), jnp.int32)]).reshape(_NT, NB, _B)
    dst = jnp.concatenate(
        [edge_index[1], jnp.full((E_pad - E,), N, jnp.int32)]).reshape(_NT, NB, _B)
    s4 = src.reshape(_NT, NB // 4, 4, 1, _B)
    d4 = dst.reshape(_NT, NB // 4, 4, 1, _B)
    packed = jnp.concatenate([s4, d4], axis=3).reshape(_NT, NB // 4, 8, _B)

    edge_kernel = _make_edge_kernel(N, NB)

    f32 = jnp.float32
    xl1, xr1 = pl.pallas_call(
        _lin_body,
        out_shape=[jax.ShapeDtypeStruct((N, 16), f32)] * 2,
    )(x, Wl1, Wr1)

    acc1 = edge_kernel(xl1, xr1, packed, att1)

    xl2, xr2 = pl.pallas_call(
        _make_mid_body(N),
        out_shape=[jax.ShapeDtypeStruct((N, 16), f32)] * 2,
    )(acc1, xl1, xr1, att1.reshape(1, 16), b1.reshape(1, 16), Wl2, Wr2)

    acc2 = edge_kernel(xl2, xr2, packed, att2)

    out = pl.pallas_call(
        _make_final_body(N),
        out_shape=jax.ShapeDtypeStruct((N, 16), f32),
    )(acc2, xl2, xr2, att2.reshape(1, 16), b2.reshape(1, 16))

    return out
